# baseline (speedup 1.0000x reference)
import functools

import jax
import jax.numpy as jnp
from jax import lax
from jax.experimental import pallas as pl
from jax.experimental.pallas import tpu as pltpu
from jax.experimental.pallas import tpu_sc as plsc

N = 1000000
G_TOTAL = N // 16
NC, NS = 2, 16
NW = NC * NS
PER = G_TOTAL // NW
REM = G_TOTAL - PER * NW
CHUNK_G = 256
NCHUNK = -(-(PER + 1) // CHUNK_G)

_mesh = plsc.VectorSubcoreMesh(core_axis_name="c", subcore_axis_name="s")


@functools.partial(
    pl.kernel,
    out_type=jax.ShapeDtypeStruct((8 * N,), jnp.float32),
    mesh=_mesh,
    scratch_types=[
        pltpu.VMEM((CHUNK_G * 48,), jnp.float32),
        pltpu.VMEM((CHUNK_G * 48,), jnp.float32),
        pltpu.VMEM((CHUNK_G * 64,), jnp.float32),
        pltpu.VMEM((CHUNK_G * 128,), jnp.float32),
    ],
    compiler_params=pltpu.CompilerParams(needs_layout_passes=False),
)
def _sc_body(n_hbm, p_hbm, f_hbm, out_hbm, n_v, p_v, f_v, o_v):
    wid = lax.axis_index("s") * NC + lax.axis_index("c")
    g0 = wid * PER + jnp.minimum(wid, REM)
    cnt = PER + jnp.where(wid < REM, 1, 0)
    g1 = g0 + cnt
    iota = lax.iota(jnp.int32, 16)
    l3 = iota * 3
    l4 = iota * 4
    l8 = iota * 8

    def frac_m05(x):
        t = x.astype(jnp.int32).astype(jnp.float32)
        t = t - jnp.where(t > x, 1.0, 0.0)
        return x - t - 0.5

    def group_body(i, _):
        bn = i * 48
        bf = i * 64
        bo = i * 128
        n0 = plsc.load_gather(n_v, [bn + l3])
        n1 = plsc.load_gather(n_v, [bn + l3 + 1])
        n2 = plsc.load_gather(n_v, [bn + l3 + 2])
        p0 = plsc.load_gather(p_v, [bn + l3])
        p1 = plsc.load_gather(p_v, [bn + l3 + 1])
        p2 = plsc.load_gather(p_v, [bn + l3 + 2])
        dis = (frac_m05(p0) * n0 + frac_m05(p1) * n1 + frac_m05(p2) * n2)
        f0 = plsc.load_gather(f_v, [bf + l4])
        f1 = plsc.load_gather(f_v, [bf + l4 + 1])
        f2 = plsc.load_gather(f_v, [bf + l4 + 2])
        f3 = plsc.load_gather(f_v, [bf + l4 + 3])
        plsc.store_scatter(o_v, [bo + l8], n0)
        plsc.store_scatter(o_v, [bo + l8 + 1], n1)
        plsc.store_scatter(o_v, [bo + l8 + 2], n2)
        plsc.store_scatter(o_v, [bo + l8 + 3], dis)
        plsc.store_scatter(o_v, [bo + l8 + 4], f0)
        plsc.store_scatter(o_v, [bo + l8 + 5], f1)
        plsc.store_scatter(o_v, [bo + l8 + 6], f2)
        plsc.store_scatter(o_v, [bo + l8 + 7], f3)
        return 0

    def chunk_body(c, _):
        gs = jnp.minimum(g0 + c * CHUNK_G, g1 - CHUNK_G)
        pltpu.sync_copy(n_hbm.at[pl.ds(gs * 48, CHUNK_G * 48)], n_v)
        pltpu.sync_copy(p_hbm.at[pl.ds(gs * 48, CHUNK_G * 48)], p_v)
        pltpu.sync_copy(f_hbm.at[pl.ds(gs * 64, CHUNK_G * 64)], f_v)
        lax.fori_loop(0, CHUNK_G, group_body, 0)
        pltpu.sync_copy(o_v, out_hbm.at[pl.ds(gs * 128, CHUNK_G * 128)])
        return 0

    lax.fori_loop(0, NCHUNK, chunk_body, 0)


def kernel(normals, points, features):
    out_flat = _sc_body(
        normals.reshape(-1), points.reshape(-1), features.reshape(-1)
    )
    return out_flat.reshape(N, 8)
